# Initial kernel scaffold; baseline (speedup 1.0000x reference)
import jax
import jax.numpy as jnp
from jax.experimental import pallas as pl


def kernel(x, W_leaf, b_leaf, W_mid, b_mid, W_root, b_root, W_gate, b_gate):
    raise NotImplementedError("write your pallas kernel here")


# speedup vs baseline: 2.2726x; 2.2726x over previous
import jax
import jax.numpy as jnp
from jax.experimental import pallas as pl

D = 1024
B = 4096
NLEAF = 4
TB = 512


def _tree_kernel(x_ref, wl_ref, bl_ref, wm_ref, bm_ref, wr_ref, br_ref,
                 wg_ref, bg_ref, out_ref):
    xt = x_ref[:]

    logits = jnp.dot(xt, wg_ref[:], preferred_element_type=jnp.float32)
    logits = logits + bg_ref[:]
    m = jnp.max(logits, axis=1, keepdims=True)
    lane = jax.lax.broadcasted_iota(jnp.int32, logits.shape, 1)
    idx = jnp.min(jnp.where(logits >= m, lane, NLEAF), axis=1,
                  keepdims=True)

    h1 = jnp.zeros((TB, D), dtype=jnp.float32)
    for e in range(NLEAF):
        a = jnp.dot(xt, wl_ref[e], preferred_element_type=jnp.float32)
        a = jnp.maximum(a + bl_ref[e:e + 1, :], 0.0)
        h1 = h1 + jnp.where(idx == e, a, 0.0)

    mask0 = idx < 2
    g0 = jnp.where(mask0, h1, 0.0)
    g1 = h1 - g0
    a0 = jnp.maximum(jnp.dot(g0, wm_ref[0], preferred_element_type=jnp.float32)
                     + bm_ref[0:1, :], 0.0)
    a1 = jnp.maximum(jnp.dot(g1, wm_ref[1], preferred_element_type=jnp.float32)
                     + bm_ref[1:2, :], 0.0)
    h2 = jnp.where(mask0, a0, 0.0) + jnp.where(mask0, 0.0, a1)

    h3 = jnp.dot(h2, wr_ref[:], preferred_element_type=jnp.float32)
    out_ref[:] = jnp.maximum(h3 + br_ref[:], 0.0)


def kernel(x, W_leaf, b_leaf, W_mid, b_mid, W_root, b_root, W_gate, b_gate):
    grid = (B // TB,)
    return pl.pallas_call(
        _tree_kernel,
        grid=grid,
        in_specs=[
            pl.BlockSpec((TB, D), lambda i: (i, 0)),
            pl.BlockSpec((NLEAF, D, D), lambda i: (0, 0, 0)),
            pl.BlockSpec((NLEAF, D), lambda i: (0, 0)),
            pl.BlockSpec((2, D, D), lambda i: (0, 0, 0)),
            pl.BlockSpec((2, D), lambda i: (0, 0)),
            pl.BlockSpec((D, D), lambda i: (0, 0)),
            pl.BlockSpec((1, D), lambda i: (0, 0)),
            pl.BlockSpec((D, NLEAF), lambda i: (0, 0)),
            pl.BlockSpec((1, NLEAF), lambda i: (0, 0)),
        ],
        out_specs=pl.BlockSpec((TB, D), lambda i: (i, 0)),
        out_shape=jax.ShapeDtypeStruct((B, D), jnp.float32),
    )(x, W_leaf, b_leaf, W_mid, b_mid, W_root,
      b_root.reshape(1, D), W_gate, b_gate.reshape(1, NLEAF))
